# Initial kernel scaffold; baseline (speedup 1.0000x reference)
import jax
import jax.numpy as jnp
from jax.experimental import pallas as pl


def kernel(batch, weight):
    raise NotImplementedError("write your pallas kernel here")


# speedup vs baseline: 1.2434x; 1.2434x over previous
import functools

import jax
import jax.numpy as jnp
from jax import lax
from jax.experimental import pallas as pl
from jax.experimental.pallas import tpu as pltpu
from jax.experimental.pallas import tpu_sc as plsc


@functools.lru_cache(maxsize=None)
def _make_gather(V, D, DP, B, C):
    info = plsc.get_sparse_core_info()
    NC, NS = info.num_cores, info.num_subcores
    NW = NC * NS
    assert B % (NW * C) == 0
    b_per_w = B // NW
    n_chunks = b_per_w // C
    mesh = plsc.VectorSubcoreMesh(core_axis_name="c", subcore_axis_name="s")

    @functools.partial(
        pl.kernel,
        mesh=mesh,
        out_type=jax.ShapeDtypeStruct((B, DP), jnp.float32),
        scratch_types=[
            pltpu.VMEM((C,), jnp.int32),
            pltpu.VMEM((C, DP), jnp.float32),
            pltpu.SemaphoreType.DMA,
        ],
    )
    def k(table_hbm, idx_hbm, out_hbm, idx_v, rows_v, sem):
        wid = lax.axis_index("s") * NC + lax.axis_index("c")
        base = wid * b_per_w

        def body(i, carry):
            off = base + i * C
            pltpu.sync_copy(idx_hbm.at[pl.ds(off, C)], idx_v)
            pltpu.async_copy(table_hbm.at[idx_v], rows_v, sem).wait()
            pltpu.sync_copy(rows_v, out_hbm.at[pl.ds(off, C)])
            return carry

        lax.fori_loop(0, n_chunks, body, 0)

    return k


def kernel(batch, weight):
    b0, b1 = batch.shape
    V, D = weight.shape
    DP = ((D + 127) // 128) * 128
    B = b0 * b1
    idx = batch.reshape(B).astype(jnp.int32)
    table = jnp.pad(weight, ((0, 0), (0, DP - D))) if DP != D else weight
    out = _make_gather(V, D, DP, B, 128)(table, idx)
    return out[:, :D].reshape(b0, b1, D)


# speedup vs baseline: 1.3648x; 1.0976x over previous
import functools

import jax
import jax.numpy as jnp
from jax import lax
from jax.experimental import pallas as pl
from jax.experimental.pallas import tpu as pltpu
from jax.experimental.pallas import tpu_sc as plsc


@functools.lru_cache(maxsize=None)
def _make_gather(V, D, DP, B, C):
    info = plsc.get_sparse_core_info()
    NC, NS = info.num_cores, info.num_subcores
    NW = NC * NS
    assert B % (NW * C) == 0
    b_per_w = B // NW
    n_chunks = b_per_w // C
    assert n_chunks % 2 == 0
    mesh = plsc.VectorSubcoreMesh(core_axis_name="c", subcore_axis_name="s")

    @functools.partial(
        pl.kernel,
        mesh=mesh,
        out_type=jax.ShapeDtypeStruct((B, DP), jnp.float32),
        scratch_types=[
            pltpu.VMEM((b_per_w,), jnp.int32),
            pltpu.VMEM((2, C, DP), jnp.float32),
            pltpu.SemaphoreType.DMA,
            pltpu.SemaphoreType.DMA,
            pltpu.SemaphoreType.DMA,
            pltpu.SemaphoreType.DMA,
        ],
    )
    def k(table_hbm, idx_hbm, out_hbm, idx_v, rows_v, sg0, sg1, so0, so1):
        wid = lax.axis_index("s") * NC + lax.axis_index("c")
        base = wid * b_per_w
        sg = (sg0, sg1)
        so = (so0, so1)

        pltpu.sync_copy(idx_hbm.at[pl.ds(base, b_per_w)], idx_v)
        pltpu.async_copy(
            table_hbm.at[idx_v.at[pl.ds(0, C)]], rows_v.at[0], sg[0]
        )

        def body(g, carry):
            for b in range(2):
                i = g + b
                cur, nxt = b, 1 - b

                @pl.when(i >= 1)
                def _():
                    pltpu.make_async_copy(
                        rows_v.at[nxt], out_hbm.at[pl.ds(base, C)], so[nxt]
                    ).wait()

                @pl.when(i + 1 < n_chunks)
                def _():
                    pltpu.async_copy(
                        table_hbm.at[idx_v.at[pl.ds((i + 1) * C, C)]],
                        rows_v.at[nxt],
                        sg[nxt],
                    )

                pltpu.make_async_copy(
                    table_hbm.at[idx_v.at[pl.ds(0, C)]], rows_v.at[cur], sg[cur]
                ).wait()
                pltpu.async_copy(
                    rows_v.at[cur], out_hbm.at[pl.ds(base + i * C, C)], so[cur]
                )
            return carry

        lax.fori_loop(0, n_chunks // 2, lambda g, c: body(2 * g, c), 0)
        pltpu.make_async_copy(
            rows_v.at[1], out_hbm.at[pl.ds(base, C)], so[1]
        ).wait()

    return k


def kernel(batch, weight):
    b0, b1 = batch.shape
    V, D = weight.shape
    DP = ((D + 127) // 128) * 128
    B = b0 * b1
    idx = batch.reshape(B).astype(jnp.int32)
    table = jnp.pad(weight, ((0, 0), (0, DP - D))) if DP != D else weight
    out = _make_gather(V, D, DP, B, 128)(table, idx)
    return out[:, :D].reshape(b0, b1, D)
